# Initial kernel scaffold; baseline (speedup 1.0000x reference)
import jax
import jax.numpy as jnp
from jax.experimental import pallas as pl


def kernel(x_upper, edge_index_upper, x_middle, edge_index_middle, x_lower, edge_index_lower, weights, W1, b1, W2, b2, W3, b3):
    raise NotImplementedError("write your pallas kernel here")


# speedup vs baseline: 11.0705x; 11.0705x over previous
import functools

import jax
import jax.numpy as jnp
from jax import lax
from jax.experimental import pallas as pl
from jax.experimental.pallas import tpu as pltpu
from jax.experimental.pallas import tpu_sc as plsc

N = 10000
E = 160000
F = 256
HALF = 128
NC = 2
NS = 16
NPAD = 10240


_E_PER_SC = E // NC
_E_PER_TILE_D = _E_PER_SC // NS
_D_CHUNKS = _E_PER_TILE_D // 128
_D_REM = _E_PER_TILE_D - _D_CHUNKS * 128


def _deg_body(dst0, dst1, dst2, const_hbm, out, deg_sp, zbuf, idx_v, idx_r,
              ones_v, ones_r, obuf):
    cid = lax.axis_index("c")
    sid = lax.axis_index("s")
    pltpu.sync_copy(const_hbm.at[pl.ds(0, 640)], zbuf)
    pltpu.sync_copy(const_hbm.at[pl.ds(640, 128)], ones_v)
    pltpu.sync_copy(const_hbm.at[pl.ds(640, 8)], ones_r)
    for g, dst in enumerate((dst0, dst1, dst2)):
        pltpu.sync_copy(zbuf, deg_sp.at[pl.ds(sid * 640, 640)])
        plsc.subcore_barrier()
        base = cid * _E_PER_SC + sid * _E_PER_TILE_D

        def chunk(i, _, dst=dst):
            pltpu.sync_copy(dst.at[pl.ds(base + i * 128, 128)], idx_v)
            pltpu.sync_copy(ones_v, deg_sp.at[idx_v], add=True)
            return _

        lax.fori_loop(0, _D_CHUNKS, chunk, 0)
        pltpu.sync_copy(dst.at[pl.ds(base + _D_CHUNKS * 128, _D_REM)], idx_r)
        pltpu.sync_copy(ones_r, deg_sp.at[idx_r], add=True)
        plsc.subcore_barrier()
        pltpu.sync_copy(deg_sp.at[pl.ds(sid * 640, 640)], obuf)
        pltpu.sync_copy(
            obuf, out.at[pl.ds(g * (NC * NPAD) + cid * NPAD + sid * 640, 640)])
        plsc.subcore_barrier()


_deg_call = pl.kernel(
    _deg_body,
    out_type=jax.ShapeDtypeStruct((3 * NC * NPAD,), jnp.float32),
    mesh=plsc.VectorSubcoreMesh(core_axis_name="c", subcore_axis_name="s"),
    scratch_types=[
        pltpu.VMEM_SHARED((NPAD,), jnp.float32),
        pltpu.VMEM((640,), jnp.float32),
        pltpu.VMEM((128,), jnp.int32),
        pltpu.VMEM((8,), jnp.int32),
        pltpu.VMEM((128,), jnp.float32),
        pltpu.VMEM((8,), jnp.float32),
        pltpu.VMEM((640,), jnp.float32),
    ],
)


_R = 1280


def _mm_body(dp_ref, x_ref, w_ref, y0_ref, y1_ref):
    dp = dp_ref[0, 0, :] + dp_ref[0, 1, :] + 1.0
    dinv = lax.rsqrt(dp)
    xw = jnp.dot(x_ref[...], w_ref[...], preferred_element_type=jnp.float32)
    y = xw * dinv[:, None]
    y0_ref[...] = y[:, :HALF]
    y1_ref[...] = y[:, HALF:]


def _mm_call(g, degp, x, W):
    return pl.pallas_call(
        _mm_body,
        grid=(pl.cdiv(N, _R),),
        in_specs=[
            pl.BlockSpec((1, NC, _R), lambda i, g=g: (g, 0, i)),
            pl.BlockSpec((_R, F), lambda i: (i, 0)),
            pl.BlockSpec((F, F), lambda i: (0, 0)),
        ],
        out_specs=[
            pl.BlockSpec((_R, HALF), lambda i: (i, 0)),
            pl.BlockSpec((_R, HALF), lambda i: (i, 0)),
        ],
        out_shape=[
            jax.ShapeDtypeStruct((N, HALF), jnp.float32),
            jax.ShapeDtypeStruct((N, HALF), jnp.float32),
        ],
    )(degp, x, W)


_E_PER_TILE = E // NS
_A_CHUNKS = _E_PER_TILE // 128
_A_REM = _E_PER_TILE - _A_CHUNKS * 128
_ROWS_PER_TILE = 624
_ROW_CHUNK = 104


def _copy_rows(src_ref, dst_ref, rowbuf, rowbuf16, sid):
    for j in range(6):
        r0 = sid * _ROWS_PER_TILE + j * _ROW_CHUNK
        pltpu.sync_copy(src_ref.at[pl.ds(r0, _ROW_CHUNK)],
                        rowbuf.at[pl.ds(0, _ROW_CHUNK)])
        pltpu.sync_copy(rowbuf.at[pl.ds(0, _ROW_CHUNK)],
                        dst_ref.at[pl.ds(r0, _ROW_CHUNK)])

    @pl.when(sid == NS - 1)
    def _():
        pltpu.sync_copy(src_ref.at[pl.ds(NS * _ROWS_PER_TILE, 16)], rowbuf16)
        pltpu.sync_copy(rowbuf16, dst_ref.at[pl.ds(NS * _ROWS_PER_TILE, 16)])


def _agg_half(y, src, dst, o, agg_sp, rowbuf, rowbuf16, sidx, didx, sidx16,
              didx16, gsem, sid):
    _copy_rows(y, agg_sp, rowbuf, rowbuf16, sid)
    plsc.subcore_barrier()
    base = sid * _E_PER_TILE

    def chunk(i, _):
        pltpu.sync_copy(src.at[pl.ds(base + i * 128, 128)], sidx)
        pltpu.sync_copy(dst.at[pl.ds(base + i * 128, 128)], didx)
        pltpu.async_copy(y.at[sidx], rowbuf, gsem).wait()
        pltpu.sync_copy(rowbuf, agg_sp.at[didx], add=True)
        return _

    lax.fori_loop(0, _A_CHUNKS, chunk, 0)
    rem0 = base + _A_CHUNKS * 128
    pltpu.sync_copy(src.at[pl.ds(rem0, _A_REM)], sidx16)
    pltpu.sync_copy(dst.at[pl.ds(rem0, _A_REM)], didx16)
    pltpu.async_copy(y.at[sidx16], rowbuf16, gsem).wait()
    pltpu.sync_copy(rowbuf16, agg_sp.at[didx16], add=True)
    plsc.subcore_barrier()
    _copy_rows(agg_sp, o, rowbuf, rowbuf16, sid)
    plsc.subcore_barrier()


def _agg_body(y00, y01, y10, y11, y20, y21, src0, dst0, src1, dst1, src2,
              dst2, a0, a1, agg_sp, rowbuf, rowbuf16, sidx, didx, sidx16,
              didx16, gsem):
    cid = lax.axis_index("c")
    sid = lax.axis_index("s")
    ys = ((y00, y01), (y10, y11), (y20, y21))
    es = ((src0, dst0), (src1, dst1), (src2, dst2))
    for g in range(3):
        src, dst = es[g]

        @pl.when(cid == 0)
        def _():
            _agg_half(ys[g][0], src, dst, a0.at[g], agg_sp, rowbuf, rowbuf16,
                      sidx, didx, sidx16, didx16, gsem, sid)

        @pl.when(cid == 1)
        def _():
            _agg_half(ys[g][1], src, dst, a1.at[g], agg_sp, rowbuf, rowbuf16,
                      sidx, didx, sidx16, didx16, gsem, sid)


_agg_call = pl.kernel(
    _agg_body,
    out_type=[
        jax.ShapeDtypeStruct((3, N, HALF), jnp.float32),
        jax.ShapeDtypeStruct((3, N, HALF), jnp.float32),
    ],
    mesh=plsc.VectorSubcoreMesh(core_axis_name="c", subcore_axis_name="s"),
    scratch_types=[
        pltpu.VMEM_SHARED((N, HALF), jnp.float32),
        pltpu.VMEM((128, HALF), jnp.float32),
        pltpu.VMEM((16, HALF), jnp.float32),
        pltpu.VMEM((128,), jnp.int32),
        pltpu.VMEM((128,), jnp.int32),
        pltpu.VMEM((16,), jnp.int32),
        pltpu.VMEM((16,), jnp.int32),
        pltpu.SemaphoreType.DMA,
    ],
)


def _ep_body(dp_ref, a0_ref, a1_ref, b_ref, w_ref, o_ref):
    g = pl.program_id(0)
    dp = dp_ref[0, 0, :] + dp_ref[0, 1, :] + 1.0
    dinv = lax.rsqrt(dp)[:, None]
    wg = w_ref[g]
    h0 = jnp.maximum(a0_ref[0] * dinv + b_ref[0, 0, :HALF][None, :], 0.0) * wg
    h1 = jnp.maximum(a1_ref[0] * dinv + b_ref[0, 0, HALF:][None, :], 0.0) * wg
    o_ref[...] = jnp.concatenate([h0, h1], axis=1)


def _ep_call(degp, a0, a1, bs, weights):
    return pl.pallas_call(
        _ep_body,
        grid=(3, pl.cdiv(N, _R)),
        in_specs=[
            pl.BlockSpec((1, NC, _R), lambda g, i: (g, 0, i)),
            pl.BlockSpec((1, _R, HALF), lambda g, i: (g, i, 0)),
            pl.BlockSpec((1, _R, HALF), lambda g, i: (g, i, 0)),
            pl.BlockSpec((1, 1, F), lambda g, i: (g, 0, 0)),
            pl.BlockSpec(memory_space=pltpu.SMEM),
        ],
        out_specs=pl.BlockSpec((_R, F), lambda g, i: (i, g)),
        out_shape=jax.ShapeDtypeStruct((N, 3 * F), jnp.float32),
    )(degp, a0, a1, bs, weights)


@jax.jit
def _run(x_upper, ei_u, x_middle, ei_m, x_lower, ei_l, weights, W1, b1, W2,
         b2, W3, b3):
    srcs = [jnp.asarray(e[0], jnp.int32) for e in (ei_u, ei_m, ei_l)]
    dsts = [jnp.asarray(e[1], jnp.int32) for e in (ei_u, ei_m, ei_l)]
    const = jnp.zeros((768,), jnp.float32).at[640:].set(1.0)
    degp = _deg_call(dsts[0], dsts[1], dsts[2], const).reshape(3, NC, NPAD)
    ys = []
    for g, (x, W) in enumerate(((x_upper, W1), (x_middle, W2),
                                (x_lower, W3))):
        ys.append(_mm_call(g, degp, x, W))
    a0, a1 = _agg_call(ys[0][0], ys[0][1], ys[1][0], ys[1][1], ys[2][0],
                       ys[2][1], srcs[0], dsts[0], srcs[1], dsts[1], srcs[2],
                       dsts[2])
    bs = jnp.stack([b1, b2, b3])[:, None, :]
    return _ep_call(degp, a0, a1, bs, weights)


def kernel(x_upper, edge_index_upper, x_middle, edge_index_middle, x_lower,
           edge_index_lower, weights, W1, b1, W2, b2, W3, b3):
    return _run(x_upper, edge_index_upper, x_middle, edge_index_middle,
                x_lower, edge_index_lower, weights, W1, b1, W2, b2, W3, b3)


# speedup vs baseline: 23.4174x; 2.1153x over previous
import jax
import jax.numpy as jnp
from jax import lax
from jax.experimental import pallas as pl
from jax.experimental.pallas import tpu as pltpu
from jax.experimental.pallas import tpu_sc as plsc

N = 10000
E = 160000
F = 256
HALF = 128
NC = 2
NS = 16
NPAD = 10240

_mesh = plsc.VectorSubcoreMesh(core_axis_name="c", subcore_axis_name="s")


_DR = 39
_DW = _DR * 128
_DCH = 3 * _DR


def _deg_body(dst0, dst1, dst2, const_hbm, out, deg_sp, zbuf, idx_all,
              idx_adj, ones_v, idx_x, obuf, lsem, ssem):
    cid = lax.axis_index("c")
    sid = lax.axis_index("s")
    wid = cid * NS + sid
    dsts = (dst0, dst1, dst2)
    for g in range(3):
        pltpu.async_copy(dsts[g].at[pl.ds(wid * _DW, _DW)],
                         idx_all.at[pl.ds(g * _DW, _DW)], lsem)
    pltpu.sync_copy(const_hbm.at[pl.ds(0, 640)], zbuf)
    pltpu.sync_copy(const_hbm.at[pl.ds(640, 128)], ones_v)
    for g in range(3):
        pltpu.sync_copy(zbuf, deg_sp.at[pl.ds(g * NPAD + sid * 640, 640)])
    for g in range(3):
        pltpu.make_async_copy(dsts[g].at[pl.ds(wid * _DW, _DW)],
                              idx_all.at[pl.ds(g * _DW, _DW)], lsem).wait()

    for g in range(3):
        def adj(k, carry, g=g):
            for j in range(8):
                v = idx_all[pl.ds(g * _DW + k * 128 + j * 16, 16)]
                idx_adj[pl.ds(g * _DW + k * 128 + j * 16, 16)] = v + g * NPAD
            return carry
        lax.fori_loop(0, _DR, adj, 0)
    plsc.subcore_barrier()

    def fire(c, carry):
        pltpu.async_copy(ones_v, deg_sp.at[idx_adj.at[pl.ds(c * 128, 128)]],
                         ssem, add=True)

        @pl.when(c >= 8)
        def _():
            pltpu.make_async_copy(
                ones_v, deg_sp.at[idx_adj.at[pl.ds(0, 128)]], ssem).wait()
        return carry

    lax.fori_loop(0, _DCH, fire, 0)
    for _ in range(8):
        pltpu.make_async_copy(ones_v, deg_sp.at[idx_adj.at[pl.ds(0, 128)]],
                              ssem).wait()

    @pl.when((cid == 0) & (sid < 2))
    def _():
        for g in range(3):
            pltpu.sync_copy(dsts[g].at[pl.ds(32 * _DW + sid * 128, 128)],
                            idx_x)
            for j in range(8):
                idx_x[pl.ds(j * 16, 16)] = (idx_x[pl.ds(j * 16, 16)]
                                            + g * NPAD)
            pltpu.sync_copy(ones_v, deg_sp.at[idx_x], add=True)

    plsc.subcore_barrier()
    for g in range(3):
        pltpu.sync_copy(deg_sp.at[pl.ds(g * NPAD + sid * 640, 640)], obuf)
        pltpu.sync_copy(
            obuf, out.at[pl.ds(g * (NC * NPAD) + cid * NPAD + sid * 640,
                               640)])


_deg_call = pl.kernel(
    _deg_body,
    out_type=jax.ShapeDtypeStruct((3 * NC * NPAD,), jnp.float32),
    mesh=_mesh,
    scratch_types=[
        pltpu.VMEM_SHARED((3 * NPAD,), jnp.float32),
        pltpu.VMEM((640,), jnp.float32),
        pltpu.VMEM((3 * _DW,), jnp.int32),
        pltpu.VMEM((3 * _DW,), jnp.int32),
        pltpu.VMEM((128,), jnp.float32),
        pltpu.VMEM((128,), jnp.int32),
        pltpu.VMEM((640,), jnp.float32),
        pltpu.SemaphoreType.DMA,
        pltpu.SemaphoreType.DMA,
    ],
)


_R = 1280


def _mm_body(dp_ref, x0, x1, x2, w0, w1, w2, y00, y01, y10, y11, y20, y21):
    xs = (x0, x1, x2)
    ws = (w0, w1, w2)
    youts = ((y00, y01), (y10, y11), (y20, y21))
    for g in range(3):
        dp = dp_ref[g, 0, :] + dp_ref[g, 1, :] + 1.0
        dinv = lax.rsqrt(dp)
        xw = jnp.dot(xs[g][...], ws[g][...],
                     preferred_element_type=jnp.float32)
        y = xw * dinv[:, None]
        youts[g][0][...] = y[:, :HALF]
        youts[g][1][...] = y[:, HALF:]


def _mm_call(degp, x_all, w_all):
    xspec = pl.BlockSpec((_R, F), lambda i: (i, 0))
    wspec = pl.BlockSpec((F, F), lambda i: (0, 0))
    yspec = pl.BlockSpec((_R, HALF), lambda i: (i, 0))
    return pl.pallas_call(
        _mm_body,
        grid=(pl.cdiv(N, _R),),
        in_specs=[pl.BlockSpec((3, NC, _R), lambda i: (0, 0, i)),
                  xspec, xspec, xspec, wspec, wspec, wspec],
        out_specs=[yspec] * 6,
        out_shape=[jax.ShapeDtypeStruct((N, HALF), jnp.float32)] * 6,
    )(degp, *x_all, *w_all)


_CH = 64
_ACH = 156
_AW = _ACH * _CH
_ROWS_PER_TILE = 624


def _copy_rows(src_ref, dst_ref, rowbuf, sid):
    for o, sz in [(k * 64, 64) for k in range(9)] + [(576, 48)]:
        r0 = sid * _ROWS_PER_TILE + o
        pltpu.sync_copy(src_ref.at[pl.ds(r0, sz)], rowbuf.at[pl.ds(0, sz)])
        pltpu.sync_copy(rowbuf.at[pl.ds(0, sz)], dst_ref.at[pl.ds(r0, sz)])

    @pl.when(sid == NS - 1)
    def _():
        pltpu.sync_copy(src_ref.at[pl.ds(NS * _ROWS_PER_TILE, 16)],
                        rowbuf.at[pl.ds(0, 16)])
        pltpu.sync_copy(rowbuf.at[pl.ds(0, 16)],
                        dst_ref.at[pl.ds(NS * _ROWS_PER_TILE, 16)])


def _agg_half(y, src, dst, o, agg_sp, bufs, sidx_all, didx_all, lsem, gs,
              ss, sid):
    pltpu.async_copy(src.at[pl.ds(sid * _AW, _AW)], sidx_all, lsem)
    pltpu.async_copy(dst.at[pl.ds(sid * _AW, _AW)], didx_all, lsem)
    _copy_rows(y, agg_sp, bufs[0], sid)
    pltpu.make_async_copy(src.at[pl.ds(sid * _AW, _AW)], sidx_all,
                          lsem).wait()
    pltpu.make_async_copy(dst.at[pl.ds(sid * _AW, _AW)], didx_all,
                          lsem).wait()
    plsc.subcore_barrier()

    def sidx(c):
        return sidx_all.at[pl.ds(c * _CH, _CH)]

    def didx(c):
        return didx_all.at[pl.ds(c * _CH, _CH)]

    def issue_g(c, b):
        pltpu.async_copy(y.at[sidx(c)], bufs[b], gs[b])

    def wait_g(c, b):
        pltpu.make_async_copy(y.at[sidx(c)], bufs[b], gs[b]).wait()

    def issue_s(c, b):
        pltpu.async_copy(bufs[b], agg_sp.at[didx(c)], ss[b], add=True)

    def wait_s(c, b):
        pltpu.make_async_copy(bufs[b], agg_sp.at[didx(c)], ss[b]).wait()

    issue_g(0, 0)

    def triple(t, carry):
        for j in range(3):
            k = 3 * t + j
            b = j
            bp = (j + 2) % 3

            @pl.when(t > 0)
            def _(k=k, b=b):
                wait_s(k - 3, b)
            if j == 0:
                @pl.when(t > 0)
                def _(k=k):
                    issue_g(k, 0)
            else:
                issue_g(k, b)
            if j == 0:
                @pl.when(t > 0)
                def _(k=k, bp=bp):
                    wait_g(k - 1, bp)
                    issue_s(k - 1, bp)
            else:
                wait_g(k - 1, bp)
                issue_s(k - 1, bp)
        return carry

    lax.fori_loop(0, _ACH // 3, triple, 0)
    wait_g(_ACH - 1, 2)
    issue_s(_ACH - 1, 2)
    wait_s(_ACH - 3, 0)
    wait_s(_ACH - 2, 1)
    wait_s(_ACH - 1, 2)

    def extra(off):
        pltpu.sync_copy(src.at[pl.ds(off, _CH)], sidx_all.at[pl.ds(0, _CH)])
        pltpu.sync_copy(dst.at[pl.ds(off, _CH)], didx_all.at[pl.ds(0, _CH)])
        pltpu.async_copy(y.at[sidx(0)], bufs[0], gs[0]).wait()
        pltpu.sync_copy(bufs[0], agg_sp.at[didx(0)], add=True)

    for s in range(4):
        @pl.when(sid == s)
        def _(s=s):
            extra(NS * _AW + s * _CH)

    plsc.subcore_barrier()
    _copy_rows(agg_sp, o, bufs[0], sid)


def _agg_body(y00, y01, y10, y11, y20, y21, src0, dst0, src1, dst1, src2,
              dst2, a00, a01, a10, a11, a20, a21, agg_sp, buf0, buf1, buf2,
              sidx_all, didx_all, lsem, g0, g1, g2, s0, s1, s2):
    cid = lax.axis_index("c")
    sid = lax.axis_index("s")
    bufs = (buf0, buf1, buf2)
    gs = (g0, g1, g2)
    ss = (s0, s1, s2)
    ys = ((y00, y01), (y10, y11), (y20, y21))
    es = ((src0, dst0), (src1, dst1), (src2, dst2))
    outs = ((a00, a01), (a10, a11), (a20, a21))
    for g in range(3):
        @pl.when(cid == 0)
        def _(g=g):
            _agg_half(ys[g][0], es[g][0], es[g][1], outs[g][0], agg_sp, bufs,
                      sidx_all, didx_all, lsem, gs, ss, sid)

        @pl.when(cid == 1)
        def _(g=g):
            _agg_half(ys[g][1], es[g][0], es[g][1], outs[g][1], agg_sp, bufs,
                      sidx_all, didx_all, lsem, gs, ss, sid)


_agg_call = pl.kernel(
    _agg_body,
    out_type=[jax.ShapeDtypeStruct((N, HALF), jnp.float32)] * 6,
    mesh=_mesh,
    scratch_types=[
        pltpu.VMEM_SHARED((N, HALF), jnp.float32),
        pltpu.VMEM((_CH, HALF), jnp.float32),
        pltpu.VMEM((_CH, HALF), jnp.float32),
        pltpu.VMEM((_CH, HALF), jnp.float32),
        pltpu.VMEM((_AW,), jnp.int32),
        pltpu.VMEM((_AW,), jnp.int32),
        pltpu.SemaphoreType.DMA,
        pltpu.SemaphoreType.DMA,
        pltpu.SemaphoreType.DMA,
        pltpu.SemaphoreType.DMA,
        pltpu.SemaphoreType.DMA,
        pltpu.SemaphoreType.DMA,
        pltpu.SemaphoreType.DMA,
    ],
)


def _ep_body(dp_ref, a00, a01, a10, a11, a20, a21, b0, b1, b2, w_ref,
             o_ref):
    halves = ((a00, a01), (a10, a11), (a20, a21))
    bs = (b0, b1, b2)
    for g in range(3):
        dp = dp_ref[g, 0, :] + dp_ref[g, 1, :] + 1.0
        dinv = lax.rsqrt(dp)[:, None]
        wg = w_ref[g]
        for h in range(2):
            a = halves[g][h][...]
            b = bs[g][h * HALF:(h + 1) * HALF][None, :]
            o_ref[:, g * F + h * HALF:g * F + (h + 1) * HALF] = (
                jnp.maximum(a * dinv + b, 0.0) * wg)


def _ep_call(degp, aggs, b1, b2, b3, weights):
    half_spec = pl.BlockSpec((_R, HALF), lambda i: (i, 0))
    bspec = pl.BlockSpec((F,), lambda i: (0,))
    return pl.pallas_call(
        _ep_body,
        grid=(pl.cdiv(N, _R),),
        in_specs=[
            pl.BlockSpec((3, NC, _R), lambda i: (0, 0, i)),
            half_spec, half_spec, half_spec, half_spec, half_spec, half_spec,
            bspec, bspec, bspec,
            pl.BlockSpec(memory_space=pltpu.SMEM),
        ],
        out_specs=pl.BlockSpec((_R, 3 * F), lambda i: (i, 0)),
        out_shape=jax.ShapeDtypeStruct((N, 3 * F), jnp.float32),
    )(degp, *aggs, b1, b2, b3, weights)


@jax.jit
def _run(x_upper, ei_u, x_middle, ei_m, x_lower, ei_l, weights, W1, b1, W2,
         b2, W3, b3):
    srcs = [jnp.asarray(e[0], jnp.int32) for e in (ei_u, ei_m, ei_l)]
    dsts = [jnp.asarray(e[1], jnp.int32) for e in (ei_u, ei_m, ei_l)]
    const = jnp.zeros((768,), jnp.float32).at[640:].set(1.0)
    degp = _deg_call(dsts[0], dsts[1], dsts[2], const).reshape(3, NC, NPAD)
    ys = _mm_call(degp, (x_upper, x_middle, x_lower), (W1, W2, W3))
    aggs = _agg_call(ys[0], ys[1], ys[2], ys[3], ys[4], ys[5],
                     srcs[0], dsts[0], srcs[1], dsts[1], srcs[2], dsts[2])
    return _ep_call(degp, aggs, b1, b2, b3, weights)


def kernel(x_upper, edge_index_upper, x_middle, edge_index_middle, x_lower,
           edge_index_lower, weights, W1, b1, W2, b2, W3, b3):
    return _run(x_upper, edge_index_upper, x_middle, edge_index_middle,
                x_lower, edge_index_lower, weights, W1, b1, W2, b2, W3, b3)
